# baseline (speedup 1.0000x reference)
import functools

import jax
import jax.numpy as jnp
from jax import lax
from jax.experimental import pallas as pl
from jax.experimental.pallas import tpu as pltpu
from jax.experimental.pallas import tpu_sc as plsc

_V = 1000000
_D = 64
_B, _L = 4096, 200
_N = _B * _L

_NC = 2
_NS = 16
_NW = _NC * _NS
_PER_W = _N // _NW
_IDXW = 128
_C = 256
_NCHUNK = _PER_W // _C

_mesh = plsc.VectorSubcoreMesh(core_axis_name="c", subcore_axis_name="s")


@functools.partial(
    pl.kernel,
    out_type=jax.ShapeDtypeStruct((_B, _L, _D), jnp.float32),
    mesh=_mesh,
    compiler_params=pltpu.CompilerParams(needs_layout_passes=False),
    scratch_types=[
        pltpu.VMEM((_C,), jnp.int32),
        pltpu.VMEM((_C,), jnp.int32),
        pltpu.VMEM((_C, 2 * _D), jnp.float32),
        pltpu.VMEM((_C, _D), jnp.float32),
        pltpu.SemaphoreType.DMA,
    ],
)
def _emb_lookup(packed, idxj_hbm, poff_hbm, out_hbm, idxj_v, poff_v,
                rows_v, rows_c, sem):
    out2 = out_hbm.reshape(_N, _D)
    wid = lax.axis_index("s") * _NC + lax.axis_index("c")
    base = wid * _PER_W

    def chunk_body(i, carry):
        off = base + i * _C
        pltpu.sync_copy(idxj_hbm.at[pl.ds(off, _C)], idxj_v)
        pltpu.sync_copy(poff_hbm.at[pl.ds(off, _C)], poff_v)
        copies = []
        for j in range(_C // _IDXW):
            copies.append(
                pltpu.async_copy(
                    packed.at[idxj_v.at[pl.ds(j * _IDXW, _IDXW)]],
                    rows_v.at[pl.ds(j * _IDXW, _IDXW)],
                    sem,
                )
            )
        for c in copies:
            c.wait()

        iota = lax.iota(jnp.int32, 16)
        m63 = jnp.full((16,), _D - 1, jnp.int32)

        def grp_body(g, carry2):
            pvec = poff_v[pl.ds(16 * g, 16)]
            rowvec = 16 * g + iota
            for c in range(_D):
                colv = (iota + jnp.full((16,), c, jnp.int32)) & m63
                vals = plsc.load_gather(rows_v, [rowvec, pvec + colv])
                plsc.store_scatter(rows_c, [rowvec, colv], vals)
            return carry2

        lax.fori_loop(0, _C // 16, grp_body, 0)
        pltpu.sync_copy(rows_c, out2.at[pl.ds(off, _C)])
        return carry

    lax.fori_loop(0, _NCHUNK, chunk_body, 0)


def kernel(y, table):
    packed = table.reshape(_V // 2, 2 * _D)
    yf = y.reshape(_N)
    idxj = yf >> 1
    poff = (yf & 1) * _D
    return _emb_lookup(packed, idxj, poff)


# speedup vs baseline: 1.1562x; 1.1562x over previous
import functools

import jax
import jax.numpy as jnp
from jax import lax
from jax.experimental import pallas as pl
from jax.experimental.pallas import tpu as pltpu
from jax.experimental.pallas import tpu_sc as plsc

_V = 1000000
_D = 64
_B, _L = 4096, 200
_N = _B * _L

_NC = 2
_NS = 16
_NW = _NC * _NS
_PER_W = _N // _NW
_IDXW = 128
_C = 256
_NCHUNK = _PER_W // _C

_mesh = plsc.VectorSubcoreMesh(core_axis_name="c", subcore_axis_name="s")


@functools.partial(
    pl.kernel,
    out_type=jax.ShapeDtypeStruct((_B, _L, _D), jnp.float32),
    mesh=_mesh,
    compiler_params=pltpu.CompilerParams(needs_layout_passes=False),
    scratch_types=[
        pltpu.VMEM((_C,), jnp.int32),
        pltpu.VMEM((_C,), jnp.int32),
        pltpu.VMEM((_C, 2 * _D), jnp.float32),
        pltpu.VMEM((_C, _D), jnp.float32),
        pltpu.SemaphoreType.DMA,
    ],
)
def _emb_lookup(packed, idxj_hbm, poff_hbm, out_hbm, idxj_v, poff_v,
                rows_v, rows_c, sem):
    out2 = out_hbm.reshape(_N, _D)
    wid = lax.axis_index("s") * _NC + lax.axis_index("c")
    base = wid * _PER_W

    def chunk_body(i, carry):
        off = base + i * _C
        pltpu.sync_copy(idxj_hbm.at[pl.ds(off, _C)], idxj_v)
        pltpu.sync_copy(poff_hbm.at[pl.ds(off, _C)], poff_v)
        copies = []
        for j in range(_C // _IDXW):
            copies.append(
                pltpu.async_copy(
                    packed.at[idxj_v.at[pl.ds(j * _IDXW, _IDXW)]],
                    rows_v.at[pl.ds(j * _IDXW, _IDXW)],
                    sem,
                )
            )
        for c in copies:
            c.wait()

        iota = lax.iota(jnp.int32, 16)
        m63 = jnp.full((16,), _D - 1, jnp.int32)

        def grp_body(g, carry2):
            pvec = poff_v[pl.ds(16 * g, 16)]
            rowvec = 16 * g + iota
            for cb in range(_D // 8):
                colvs, valss = [], []
                for c in range(8 * cb, 8 * cb + 8):
                    colv = (iota + jnp.full((16,), c, jnp.int32)) & m63
                    colvs.append(colv)
                    valss.append(
                        plsc.load_gather(rows_v, [rowvec, pvec + colv]))
                for colv, vals in zip(colvs, valss):
                    plsc.store_scatter(rows_c, [rowvec, colv], vals)
            return carry2

        lax.fori_loop(0, _C // 16, grp_body, 0)
        pltpu.sync_copy(rows_c, out2.at[pl.ds(off, _C)])
        return carry

    lax.fori_loop(0, _NCHUNK, chunk_body, 0)


def kernel(y, table):
    packed = table.reshape(_V // 2, 2 * _D)
    yf = y.reshape(_N)
    idxj = yf >> 1
    poff = (yf & 1) * _D
    return _emb_lookup(packed, idxj, poff)


# speedup vs baseline: 1.4615x; 1.2640x over previous
import functools

import jax
import jax.numpy as jnp
from jax import lax
from jax.experimental import pallas as pl
from jax.experimental.pallas import tpu as pltpu
from jax.experimental.pallas import tpu_sc as plsc

_V = 1000000
_D = 64
_B, _L = 4096, 200
_N = _B * _L

_NC = 2
_NS = 16
_NW = _NC * _NS
_PER_W = _N // _NW
_C = 128
_NCHUNK = _PER_W // _C

_mesh = plsc.VectorSubcoreMesh(core_axis_name="c", subcore_axis_name="s")


@functools.partial(
    pl.kernel,
    out_type=jax.ShapeDtypeStruct((_B, _L, _D), jnp.float32),
    mesh=_mesh,
    compiler_params=pltpu.CompilerParams(needs_layout_passes=False),
    scratch_types=[
        pltpu.VMEM((_PER_W,), jnp.int32),
        pltpu.VMEM((2, _C), jnp.int32),
        pltpu.VMEM((2, _C, 2 * _D), jnp.float32),
        pltpu.VMEM((2, _C, _D), jnp.float32),
        pltpu.SemaphoreType.DMA,
        pltpu.SemaphoreType.DMA,
        pltpu.SemaphoreType.DMA,
        pltpu.SemaphoreType.DMA,
    ],
)
def _emb_lookup(packed, idx_hbm, out_hbm, idx_v, idxj_v, rows_v, rows_c,
                sg0, sg1, sw0, sw1):
    out2 = out_hbm.reshape(_N, _D)
    wid = lax.axis_index("s") * _NC + lax.axis_index("c")
    base = wid * _PER_W
    iota = lax.iota(jnp.int32, 16)
    one = jnp.full((16,), 1, jnp.int32)
    m63 = jnp.full((16,), _D - 1, jnp.int32)
    hw = jnp.full((16,), _D, jnp.int32)
    gsems = (sg0, sg1)
    wsems = (sw0, sw1)

    pltpu.sync_copy(idx_hbm.at[pl.ds(base, _PER_W)], idx_v)

    def start_gather(i, b):
        for m in range(_C // 16):
            v = idx_v[pl.ds(i * _C + 16 * m, 16)]
            idxj_v[b, pl.ds(16 * m, 16)] = lax.shift_right_logical(v, one)
        pltpu.async_copy(packed.at[idxj_v.at[b]], rows_v.at[b], gsems[b])

    def wait_gather(b):
        pltpu.make_async_copy(
            packed.at[idxj_v.at[b]], rows_v.at[b], gsems[b]).wait()

    def compact(i, b):
        def grp_body(g, carry):
            yv = idx_v[pl.ds(i * _C + 16 * g, 16)]
            pvec = (yv & one) * hw
            rowvec = 16 * g + iota
            for cb in range(_D // 8):
                colvs, valss = [], []
                for c in range(8 * cb, 8 * cb + 8):
                    colv = (iota + jnp.full((16,), c, jnp.int32)) & m63
                    colvs.append(colv)
                    valss.append(
                        plsc.load_gather(rows_v.at[b], [rowvec, pvec + colv]))
                for colv, vals in zip(colvs, valss):
                    plsc.store_scatter(rows_c.at[b], [rowvec, colv], vals)
            return carry

        lax.fori_loop(0, _C // 16, grp_body, 0)

    def start_write(i, b):
        pltpu.async_copy(
            rows_c.at[b], out2.at[pl.ds(base + i * _C, _C)], wsems[b])

    def wait_write(i, b):
        pltpu.make_async_copy(
            rows_c.at[b], out2.at[pl.ds(base + i * _C, _C)], wsems[b]).wait()

    start_gather(0, 0)

    def pair_body(io, carry):
        for b in range(2):
            i = 2 * io + b

            @pl.when(i + 1 < _NCHUNK)
            def _():
                start_gather(i + 1, 1 - b)

            wait_gather(b)

            @pl.when(i >= 2)
            def _():
                wait_write(i - 2, b)

            compact(i, b)
            start_write(i, b)
        return carry

    lax.fori_loop(0, _NCHUNK // 2, pair_body, 0)
    wait_write(_NCHUNK - 2, 0)
    wait_write(_NCHUNK - 1, 1)


def kernel(y, table):
    packed = table.reshape(_V // 2, 2 * _D)
    idx = y.reshape(_N)
    return _emb_lookup(packed, idx)
